# baseline (speedup 1.0000x reference)
import functools

import jax
import jax.numpy as jnp
from jax import lax
from jax.experimental import pallas as pl
from jax.experimental.pallas import tpu as pltpu
from jax.experimental.pallas import tpu_sc as plsc

N_STARS = 1000000
BATCH = 16384
N_FREQ = 200
EPS = 0.001

_NC = 2
_NS = 16
_NW = _NC * _NS
_BPW = BATCH // _NW
_L = 16
_IDXW = 128
_NJ = _BPW // _IDXW


def _delta_gather_body(idx_hbm, hard_hbm, corr_hbm, out_hbm, idx_v, h_v, c_v, sem):
    wid = lax.axis_index("s") * _NC + lax.axis_index("c")
    pltpu.sync_copy(idx_hbm.at[wid], idx_v)
    cps = []
    for j in range(_NJ):
        cps.append(pltpu.async_copy(hard_hbm.at[idx_v.at[j]], h_v.at[j], sem))
        cps.append(pltpu.async_copy(corr_hbm.at[idx_v.at[j]], c_v.at[j], sem))
    for cp in cps:
        cp.wait()
    eps = jnp.float32(EPS)
    for j in range(_NJ):
        for i in range(_IDXW // _L):
            s = pl.ds(i * _L, _L)
            h_v[j, s] = jnp.maximum(h_v[j, s] + c_v[j, s], eps)
    pltpu.sync_copy(h_v, out_hbm.at[wid])


@jax.jit
def _gather_delta(star_indices, hard, corr):
    idx3 = star_indices.reshape(_NW, _NJ, _IDXW)
    mesh = plsc.VectorSubcoreMesh(core_axis_name="c", subcore_axis_name="s")
    fn = pl.kernel(
        _delta_gather_body,
        mesh=mesh,
        out_type=jax.ShapeDtypeStruct((_NW, _NJ, _IDXW), jnp.float32),
        scratch_types=[
            pltpu.VMEM((_NJ, _IDXW), jnp.int32),
            pltpu.VMEM((_NJ, _IDXW), jnp.float32),
            pltpu.VMEM((_NJ, _IDXW), jnp.float32),
            pltpu.SemaphoreType.DMA,
        ],
    )
    return fn(idx3, hard, corr).reshape(BATCH)


_BR = 1024
_G = BATCH // _BR
_NBUF = 4


def _mod_body(d_ref, f_hbm, o_hbm, ibuf, obuf, isem, osem):
    i = pl.program_id(0)
    slot = lax.rem(i, _NBUF)

    @pl.when(i == 0)
    def _warmup():
        for b in range(_NBUF):
            pltpu.make_async_copy(
                f_hbm.at[pl.ds(b * _BR, _BR)], ibuf.at[b], isem.at[b]
            ).start()

    pltpu.make_async_copy(
        f_hbm.at[pl.ds(i * _BR, _BR)], ibuf.at[slot], isem.at[slot]
    ).wait()

    @pl.when(i >= _NBUF)
    def _wait_out():
        pltpu.make_async_copy(
            obuf.at[slot], o_hbm.at[pl.ds((i - _NBUF) * _BR, _BR)], osem.at[slot]
        ).wait()

    obuf[slot] = jnp.mod(ibuf[slot], d_ref[...])

    pltpu.make_async_copy(
        obuf.at[slot], o_hbm.at[pl.ds(i * _BR, _BR)], osem.at[slot]
    ).start()

    @pl.when(i + _NBUF < _G)
    def _prefetch():
        pltpu.make_async_copy(
            f_hbm.at[pl.ds((i + _NBUF) * _BR, _BR)], ibuf.at[slot], isem.at[slot]
        ).start()

    @pl.when(i == _G - 1)
    def _drain():
        for b in range(_NBUF):
            j = _G - _NBUF + b
            pltpu.make_async_copy(
                obuf.at[lax.rem(j, _NBUF)],
                o_hbm.at[pl.ds(j * _BR, _BR)],
                osem.at[lax.rem(j, _NBUF)],
            ).wait()


@jax.jit
def _apply_mod(frequencies, delta):
    return pl.pallas_call(
        _mod_body,
        grid=(_G,),
        in_specs=[
            pl.BlockSpec((_BR, 1), lambda i: (i, 0)),
            pl.BlockSpec(memory_space=pltpu.MemorySpace.HBM),
        ],
        out_specs=pl.BlockSpec(memory_space=pltpu.MemorySpace.HBM),
        out_shape=jax.ShapeDtypeStruct((BATCH, N_FREQ), jnp.float32),
        scratch_shapes=[
            pltpu.VMEM((_NBUF, _BR, N_FREQ), jnp.float32),
            pltpu.VMEM((_NBUF, _BR, N_FREQ), jnp.float32),
            pltpu.SemaphoreType.DMA((_NBUF,)),
            pltpu.SemaphoreType.DMA((_NBUF,)),
        ],
    )(delta[:, None], frequencies)


_FL = BATCH * N_FREQ
_FCH = _FL // 16


def _flat_body(f_ref, o_ref):
    o_ref[...] = f_ref[...] + jnp.float32(1.0)


@jax.jit
def _flat_probe(frequencies):
    ff = frequencies.reshape(_FL)
    out = pl.pallas_call(
        _flat_body,
        grid=(16,),
        in_specs=[pl.BlockSpec((_FCH,), lambda i: (i,))],
        out_specs=pl.BlockSpec((_FCH,), lambda i: (i,)),
        out_shape=jax.ShapeDtypeStruct((_FL,), jnp.float32),
    )(ff)
    return out.reshape(BATCH, N_FREQ)


def kernel(frequencies, star_indices, delta_nu_hard, delta_nu_corr):
    return _flat_probe(frequencies)


# speedup vs baseline: 2.4384x; 2.4384x over previous
import jax
import jax.numpy as jnp
from jax import lax
from jax.experimental import pallas as pl
from jax.experimental.pallas import tpu as pltpu
from jax.experimental.pallas import tpu_sc as plsc

N_STARS = 1000000
BATCH = 16384
N_FREQ = 200
EPS = 0.001

_NC = 2
_NS = 16
_NW = _NC * _NS
_BPW = BATCH // _NW
_L = 16
_IDXW = 128
_NJ = _BPW // _IDXW


def _delta_gather_body(idx_hbm, hard_hbm, corr_hbm, out_hbm, idx_v, h_v, c_v, sem):
    wid = lax.axis_index("s") * _NC + lax.axis_index("c")
    base = wid * _BPW
    for j in range(_NJ):
        pltpu.sync_copy(idx_hbm.at[pl.ds(base + j * _IDXW, _IDXW)], idx_v.at[j])
    cps = []
    for j in range(_NJ):
        cps.append(pltpu.async_copy(hard_hbm.at[idx_v.at[j]], h_v.at[j], sem))
        cps.append(pltpu.async_copy(corr_hbm.at[idx_v.at[j]], c_v.at[j], sem))
    for cp in cps:
        cp.wait()
    eps = jnp.float32(EPS)
    for j in range(_NJ):
        for i in range(_IDXW // _L):
            s = pl.ds(i * _L, _L)
            h_v[j, s] = jnp.maximum(h_v[j, s] + c_v[j, s], eps)
    for j in range(_NJ):
        pltpu.sync_copy(h_v.at[j], out_hbm.at[pl.ds(base + j * _IDXW, _IDXW)])


def _gather_delta(star_indices, hard, corr):
    mesh = plsc.VectorSubcoreMesh(core_axis_name="c", subcore_axis_name="s")
    fn = pl.kernel(
        _delta_gather_body,
        mesh=mesh,
        out_type=jax.ShapeDtypeStruct((BATCH,), jnp.float32),
        scratch_types=[
            pltpu.VMEM((_NJ, _IDXW), jnp.int32),
            pltpu.VMEM((_NJ, _IDXW), jnp.float32),
            pltpu.VMEM((_NJ, _IDXW), jnp.float32),
            pltpu.SemaphoreType.DMA,
        ],
    )
    return fn(star_indices, hard, corr)


_BS = 8
_GT = N_FREQ // _BS


def _mod_body(d_ref, f_ref, o_ref):
    o_ref[...] = jnp.mod(f_ref[...], d_ref[...][None, :])


def _apply_mod_t(freq_t, delta):
    return pl.pallas_call(
        _mod_body,
        grid=(_GT,),
        in_specs=[
            pl.BlockSpec((BATCH,), lambda i: (0,)),
            pl.BlockSpec((_BS, BATCH), lambda i: (i, 0)),
        ],
        out_specs=pl.BlockSpec((_BS, BATCH), lambda i: (i, 0)),
        out_shape=jax.ShapeDtypeStruct((N_FREQ, BATCH), jnp.float32),
    )(delta, freq_t)


def kernel(frequencies, star_indices, delta_nu_hard, delta_nu_corr):
    idx = star_indices.astype(jnp.int32)
    delta = _gather_delta(idx, delta_nu_hard, delta_nu_corr)
    out_t = _apply_mod_t(frequencies.T, delta)
    return out_t.T


# speedup vs baseline: 3.0161x; 1.2369x over previous
import jax
import jax.numpy as jnp
from jax import lax
from jax.experimental import pallas as pl
from jax.experimental.pallas import tpu as pltpu
from jax.experimental.pallas import tpu_sc as plsc

N_STARS = 1000000
BATCH = 16384
N_FREQ = 200
EPS = 0.001

_NC = 2
_NS = 16
_NW = _NC * _NS
_BPW = BATCH // _NW
_L = 16
_IDXW = 128
_NJ = _BPW // _IDXW


def _delta_gather_body(idx_hbm, hard_hbm, corr_hbm, out_hbm, idx_v, h_v, c_v, sem):
    wid = lax.axis_index("s") * _NC + lax.axis_index("c")
    base = wid * _BPW
    for j in range(_NJ):
        pltpu.sync_copy(idx_hbm.at[pl.ds(base + j * _IDXW, _IDXW)], idx_v.at[j])
    cps = []
    for j in range(_NJ):
        cps.append(pltpu.async_copy(hard_hbm.at[idx_v.at[j]], h_v.at[j], sem))
        cps.append(pltpu.async_copy(corr_hbm.at[idx_v.at[j]], c_v.at[j], sem))
    for cp in cps:
        cp.wait()
    eps = jnp.float32(EPS)
    for j in range(_NJ):
        for i in range(_IDXW // _L):
            s = pl.ds(i * _L, _L)
            h_v[j, s] = jnp.maximum(h_v[j, s] + c_v[j, s], eps)
    for j in range(_NJ):
        pltpu.sync_copy(h_v.at[j], out_hbm.at[pl.ds(base + j * _IDXW, _IDXW)])


def _gather_delta(star_indices, hard, corr):
    mesh = plsc.VectorSubcoreMesh(core_axis_name="c", subcore_axis_name="s")
    fn = pl.kernel(
        _delta_gather_body,
        mesh=mesh,
        out_type=jax.ShapeDtypeStruct((BATCH,), jnp.float32),
        scratch_types=[
            pltpu.VMEM((_NJ, _IDXW), jnp.int32),
            pltpu.VMEM((_NJ, _IDXW), jnp.float32),
            pltpu.VMEM((_NJ, _IDXW), jnp.float32),
            pltpu.SemaphoreType.DMA,
        ],
    )
    return fn(star_indices, hard, corr)


_BS = 40
_GT = N_FREQ // _BS


def _mod_body(d_ref, f_ref, o_ref):
    o_ref[...] = jnp.mod(f_ref[...], d_ref[...][None, :])


def _apply_mod_t(freq_t, delta):
    return pl.pallas_call(
        _mod_body,
        grid=(_GT,),
        in_specs=[
            pl.BlockSpec((BATCH,), lambda i: (0,)),
            pl.BlockSpec((_BS, BATCH), lambda i: (i, 0)),
        ],
        out_specs=pl.BlockSpec((_BS, BATCH), lambda i: (i, 0)),
        out_shape=jax.ShapeDtypeStruct((N_FREQ, BATCH), jnp.float32),
    )(delta, freq_t)


def kernel(frequencies, star_indices, delta_nu_hard, delta_nu_corr):
    idx = star_indices.astype(jnp.int32)
    delta = _gather_delta(idx, delta_nu_hard, delta_nu_corr)
    out_t = _apply_mod_t(frequencies.T, delta)
    return out_t.T
